# Initial kernel scaffold; baseline (speedup 1.0000x reference)
import jax
import jax.numpy as jnp
from jax.experimental import pallas as pl


def kernel(x, edge_index, edge_type, norm, w_loop, w_bases, w_rel, w_bias, bn_gamma, bn_beta):
    raise NotImplementedError("write your pallas kernel here")


# speedup vs baseline: 3.4845x; 3.4845x over previous
import functools

import jax
import jax.numpy as jnp
from jax import lax
from jax.experimental import pallas as pl
from jax.experimental.pallas import tpu as pltpu
from jax.experimental.pallas import tpu_sc as plsc

_NC = 2
_NS = 16
_L = 16
_G = 64
_NREP = 128
_EPS = 1e-5


def _matmul_call(x, w_cat, w_loop, n, d_in, d_cat, d_out, nb):
  def body(x_ref, wc_ref, wl_ref, h_ref, lm_ref):
    xv = x_ref[...]
    h_ref[...] = jnp.dot(xv, wc_ref[...], preferred_element_type=jnp.float32)
    lm_ref[...] = jnp.dot(xv, wl_ref[...], preferred_element_type=jnp.float32)

  grid = (n // nb,)
  return pl.pallas_call(
      body,
      grid=grid,
      in_specs=[
          pl.BlockSpec((nb, d_in), lambda i: (i, 0)),
          pl.BlockSpec((d_in, d_cat), lambda i: (0, 0)),
          pl.BlockSpec((d_in, d_out), lambda i: (0, 0)),
      ],
      out_specs=[
          pl.BlockSpec((nb, d_cat), lambda i: (i, 0)),
          pl.BlockSpec((nb, d_out), lambda i: (i, 0)),
      ],
      out_shape=[
          jax.ShapeDtypeStruct((n, d_cat), jnp.float32),
          jax.ShapeDtypeStruct((n, d_out), jnp.float32),
      ],
  )(x, w_cat, w_loop)


def _edge_call(h_cat, srcg, dstg, etg, wr_rep, n_pad, d_out, n_bases, gpw):
  d_cat = n_bases * d_out
  rows_ps = n_pad // _NS
  assert rows_ps % _G == 0
  nz = rows_ps // _G
  nvec = d_out // _L

  mesh = plsc.VectorSubcoreMesh(
      core_axis_name="c", subcore_axis_name="s",
      num_cores=_NC, num_subcores=_NS)

  @functools.partial(
      pl.kernel,
      out_type=jax.ShapeDtypeStruct((_NC, n_pad, d_out), jnp.float32),
      mesh=mesh,
      scratch_types=[
          pltpu.VMEM((_G,), jnp.int32),
          pltpu.VMEM((_G,), jnp.int32),
          pltpu.VMEM((_G,), jnp.int32),
          pltpu.VMEM((_G, d_cat), jnp.float32),
          pltpu.VMEM((_G, d_out), jnp.float32),
          pltpu.VMEM((_NREP * n_bases * _L,), jnp.float32),
          pltpu.VMEM_SHARED((n_pad, d_out), jnp.float32),
          pltpu.SemaphoreType.DMA,
      ],
  )
  def edge_kernel(h_hbm, src_hbm, dst_hbm, et_hbm, wr_hbm, out_hbm,
                  src_v, dst_v, et_v, rows_v, msg_v, wr_v, agg_sp,
                  sem):
    c = lax.axis_index("c")
    s = lax.axis_index("s")

    pltpu.sync_copy(wr_hbm, wr_v)

    def zrow(r, _):
      for j in range(d_out // _L):
        msg_v[r, pl.ds(j * _L, _L)] = jnp.zeros((_L,), jnp.float32)
      return 0
    lax.fori_loop(0, _G, zrow, 0)
    zbase = s * rows_ps
    for i in range(nz):
      pltpu.sync_copy(msg_v, agg_sp.at[pl.ds(zbase + i * _G, _G)])
    plsc.subcore_barrier()

    base_g = (c * _NS + s) * gpw

    def group_body(g, _):
      gi = base_g + g
      pltpu.sync_copy(src_hbm.at[gi], src_v)
      pltpu.sync_copy(dst_hbm.at[gi], dst_v)
      pltpu.sync_copy(et_hbm.at[gi], et_v)
      pltpu.async_copy(h_hbm.at[src_v], rows_v, sem).wait()

      def sub_body(k, _):
        e0 = k * _L
        et16 = et_v[pl.ds(e0, _L)]
        for l in range(_L):
          ei = e0 + l
          off = et16[l] * (n_bases * _L)
          cb = [wr_v[pl.ds(off + b * _L, _L)] for b in range(n_bases)]
          for j in range(nvec):
            acc = cb[0] * rows_v[ei, pl.ds(j * _L, _L)]
            for b in range(1, n_bases):
              acc = acc + cb[b] * rows_v[ei, pl.ds(b * d_out + j * _L, _L)]
            msg_v[ei, pl.ds(j * _L, _L)] = acc
        return 0
      lax.fori_loop(0, _G // _L, sub_body, 0)

      pltpu.sync_copy(msg_v, agg_sp.at[dst_v], add=True)
      return 0
    lax.fori_loop(0, gpw, group_body, 0)

    plsc.subcore_barrier()
    pltpu.sync_copy(agg_sp.at[pl.ds(zbase, rows_ps)],
                    out_hbm.at[c, pl.ds(zbase, rows_ps)])

  return edge_kernel(h_cat, srcg, dstg, etg, wr_rep)


def _epilogue_call(agg2, loop_msg, norm, bias, gamma, beta, n, d_out, nb):
  nblocks = n // nb

  def body(agg_ref, lm_ref, norm_ref, b_ref, g_ref, be_ref, o_ref,
           pre_ref, sum_ref, sq_ref):
    ph = pl.program_id(0)
    i = pl.program_id(1)

    @pl.when(ph == 0)
    def _compute():
      a = agg_ref[0] + agg_ref[1]
      pre = a * norm_ref[...] + b_ref[...] + lm_ref[...]
      pre_ref[pl.ds(i * nb, nb), :] = pre

      @pl.when(i == 0)
      def _init():
        sum_ref[...] = jnp.zeros_like(sum_ref)
        sq_ref[...] = jnp.zeros_like(sq_ref)

      sum_ref[...] += jnp.sum(pre, axis=0, keepdims=True)
      sq_ref[...] += jnp.sum(pre * pre, axis=0, keepdims=True)

    @pl.when(ph == 1)
    def _normalize():
      inv_n = 1.0 / n
      mean = sum_ref[...] * inv_n
      var = sq_ref[...] * inv_n - mean * mean
      scale = g_ref[...] * lax.rsqrt(var + _EPS)
      shift = be_ref[...] - mean * scale
      pre = pre_ref[pl.ds(i * nb, nb), :]
      o_ref[...] = jnp.maximum(pre * scale + shift, 0.0)

  return pl.pallas_call(
      body,
      grid=(2, nblocks),
      in_specs=[
          pl.BlockSpec((2, nb, d_out), lambda p, i: (0, i, 0)),
          pl.BlockSpec((nb, d_out), lambda p, i: (i, 0)),
          pl.BlockSpec((nb, 1), lambda p, i: (i, 0)),
          pl.BlockSpec((1, d_out), lambda p, i: (0, 0)),
          pl.BlockSpec((1, d_out), lambda p, i: (0, 0)),
          pl.BlockSpec((1, d_out), lambda p, i: (0, 0)),
      ],
      out_specs=pl.BlockSpec((nb, d_out), lambda p, i: (i, 0)),
      out_shape=jax.ShapeDtypeStruct((n, d_out), jnp.float32),
      scratch_shapes=[
          pltpu.VMEM((n, d_out), jnp.float32),
          pltpu.VMEM((1, d_out), jnp.float32),
          pltpu.VMEM((1, d_out), jnp.float32),
      ],
  )(agg2, loop_msg, norm, bias, gamma, beta)


def kernel(x, edge_index, edge_type, norm, w_loop, w_bases, w_rel,
           w_bias, bn_gamma, bn_beta):
  n, d_in = x.shape
  e = edge_index.shape[1]
  n_bases, _, d_out = w_bases.shape
  n_rel = w_rel.shape[0]

  w_cat = jnp.concatenate([w_bases[b] for b in range(n_bases)], axis=1)

  workers = _NC * _NS
  ng = -(-e // (_G * workers)) * workers
  gpw = ng // workers
  e_pad = ng * _G
  pad = e_pad - e
  src = jnp.concatenate([edge_index[0], jnp.zeros((pad,), jnp.int32)])
  dst = jnp.concatenate([edge_index[1], jnp.zeros((pad,), jnp.int32)])
  et = jnp.concatenate([edge_type, jnp.full((pad,), n_rel, jnp.int32)])
  srcg = src.reshape(ng, _G)
  dstg = dst.reshape(ng, _G)
  etg = et.reshape(ng, _G)
  wr_rep = (jnp.zeros((_NREP, n_bases, _L), jnp.float32)
            .at[:n_rel].set(jnp.broadcast_to(w_rel[:, :, None],
                                             (n_rel, n_bases, _L)))
            .reshape(_NREP * n_bases * _L))

  n_pad = -(-n // (_G * _NS)) * (_G * _NS)
  h_cat, loop_msg = _matmul_call(
      x, w_cat, w_loop, n, d_in, n_bases * d_out, d_out, nb=2000)
  agg2 = _edge_call(h_cat, srcg, dstg, etg, wr_rep, n_pad, d_out, n_bases, gpw)
  out = _epilogue_call(
      agg2, loop_msg, norm, w_bias.reshape(1, d_out),
      bn_gamma.reshape(1, d_out), bn_beta.reshape(1, d_out), n, d_out,
      nb=2000)
  return out


# speedup vs baseline: 4.9804x; 1.4293x over previous
import functools

import jax
import jax.numpy as jnp
from jax import lax
from jax.experimental import pallas as pl
from jax.experimental.pallas import tpu as pltpu
from jax.experimental.pallas import tpu_sc as plsc

_NC = 2
_NS = 16
_L = 16
_G = 64
_NREP = 128
_EPS = 1e-5


def _matmul_call(x, w_cat, w_loop, n, d_in, d_cat, d_out, nb):
  def body(x_ref, wc_ref, wl_ref, h_ref, lm_ref):
    xv = x_ref[...]
    h_ref[...] = jnp.dot(xv, wc_ref[...], preferred_element_type=jnp.float32)
    lm_ref[...] = jnp.dot(xv, wl_ref[...], preferred_element_type=jnp.float32)

  grid = (n // nb,)
  return pl.pallas_call(
      body,
      grid=grid,
      in_specs=[
          pl.BlockSpec((nb, d_in), lambda i: (i, 0)),
          pl.BlockSpec((d_in, d_cat), lambda i: (0, 0)),
          pl.BlockSpec((d_in, d_out), lambda i: (0, 0)),
      ],
      out_specs=[
          pl.BlockSpec((nb, d_cat), lambda i: (i, 0)),
          pl.BlockSpec((nb, d_out), lambda i: (i, 0)),
      ],
      out_shape=[
          jax.ShapeDtypeStruct((n, d_cat), jnp.float32),
          jax.ShapeDtypeStruct((n, d_out), jnp.float32),
      ],
  )(x, w_cat, w_loop)


def _edge_call(h_cat, srcg, dstg, etg, wr_rep, n_pad, d_out, n_bases, gpw):
  d_cat = n_bases * d_out
  rows_ps = n_pad // _NS
  assert rows_ps % _G == 0
  nz = rows_ps // _G
  assert gpw % 2 == 0
  ng_total = gpw * _NC * _NS
  nvec = d_out // _L

  mesh = plsc.VectorSubcoreMesh(
      core_axis_name="c", subcore_axis_name="s",
      num_cores=_NC, num_subcores=_NS)

  @functools.partial(
      pl.kernel,
      out_type=jax.ShapeDtypeStruct((_NC, n_pad, d_out), jnp.float32),
      mesh=mesh,
      scratch_types=[
          [pltpu.VMEM((_G,), jnp.int32)] * 2,
          [pltpu.VMEM((_G,), jnp.int32)] * 2,
          [pltpu.VMEM((_G,), jnp.int32)] * 2,
          [pltpu.VMEM((_G, d_cat), jnp.float32)] * 2,
          pltpu.VMEM((_G, d_out), jnp.float32),
          pltpu.VMEM((_NREP * n_bases * _L,), jnp.float32),
          pltpu.VMEM_SHARED((n_pad, d_out), jnp.float32),
          [pltpu.SemaphoreType.DMA] * 2,
          [pltpu.SemaphoreType.DMA] * 2,
      ],
  )
  def edge_kernel(h_hbm, src_hbm, dst_hbm, et_hbm, wr_hbm, out_hbm,
                  src_v, dst_v, et_v, rows_v, msg_v, wr_v, agg_sp,
                  isem, gsem):
    c = lax.axis_index("c")
    s = lax.axis_index("s")

    pltpu.sync_copy(wr_hbm, wr_v)

    def zrow(r, _):
      for j in range(d_out // _L):
        msg_v[r, pl.ds(j * _L, _L)] = jnp.zeros((_L,), jnp.float32)
      return 0
    lax.fori_loop(0, _G, zrow, 0)
    zbase = s * rows_ps
    for i in range(nz):
      pltpu.sync_copy(msg_v, agg_sp.at[pl.ds(zbase + i * _G, _G)])
    plsc.subcore_barrier()

    base_g = (c * _NS + s) * gpw
    last_g = ng_total - 1

    def issue_idx(b, gi):
      gi = jnp.minimum(gi, last_g)
      pltpu.async_copy(src_hbm.at[gi], src_v[b], isem[b])
      pltpu.async_copy(dst_hbm.at[gi], dst_v[b], isem[b])
      pltpu.async_copy(et_hbm.at[gi], et_v[b], isem[b])

    def wait_idx(b):
      pltpu.make_async_copy(src_hbm.at[0], src_v[b], isem[b]).wait()
      pltpu.make_async_copy(dst_hbm.at[0], dst_v[b], isem[b]).wait()
      pltpu.make_async_copy(et_hbm.at[0], et_v[b], isem[b]).wait()

    def issue_gather(b):
      pltpu.async_copy(h_hbm.at[src_v[b]], rows_v[b], gsem[b])

    def wait_gather(b):
      pltpu.make_async_copy(h_hbm.at[src_v[b]], rows_v[b], gsem[b]).wait()

    def process(b):
      rv = rows_v[b]

      def sub_body(k, _):
        e0 = k * _L
        et16 = et_v[b][pl.ds(e0, _L)]
        for l in range(_L):
          ei = e0 + l
          off = et16[l] * (n_bases * _L)
          cb = [wr_v[pl.ds(off + bb * _L, _L)] for bb in range(n_bases)]
          for j in range(nvec):
            acc = cb[0] * rv[ei, pl.ds(j * _L, _L)]
            for bb in range(1, n_bases):
              acc = acc + cb[bb] * rv[ei, pl.ds(bb * d_out + j * _L, _L)]
            msg_v[ei, pl.ds(j * _L, _L)] = acc
        return 0
      lax.fori_loop(0, _G // _L, sub_body, 0)
      pltpu.sync_copy(msg_v, agg_sp.at[dst_v[b]], add=True)

    issue_idx(0, base_g)
    issue_idx(1, base_g + 1)
    wait_idx(0)
    issue_gather(0)

    def pipe_body(t, _):
      g = base_g + 2 * t
      wait_idx(1)
      issue_gather(1)
      wait_gather(0)
      process(0)
      issue_idx(0, g + 2)
      wait_idx(0)
      issue_gather(0)
      wait_gather(1)
      process(1)
      issue_idx(1, g + 3)
      return 0
    lax.fori_loop(0, gpw // 2, pipe_body, 0)

    wait_gather(0)
    wait_idx(1)

    plsc.subcore_barrier()
    pltpu.sync_copy(agg_sp.at[pl.ds(zbase, rows_ps)],
                    out_hbm.at[c, pl.ds(zbase, rows_ps)])

  return edge_kernel(h_cat, srcg, dstg, etg, wr_rep)


def _epilogue_call(agg2, loop_msg, norm, bias, gamma, beta, n, d_out, nb):
  nblocks = n // nb

  def body(agg_ref, lm_ref, norm_ref, b_ref, g_ref, be_ref, o_ref,
           pre_ref, sum_ref, sq_ref):
    ph = pl.program_id(0)
    i = pl.program_id(1)

    @pl.when(ph == 0)
    def _compute():
      a = agg_ref[0] + agg_ref[1]
      pre = a * norm_ref[...] + b_ref[...] + lm_ref[...]
      pre_ref[pl.ds(i * nb, nb), :] = pre

      @pl.when(i == 0)
      def _init():
        sum_ref[...] = jnp.zeros_like(sum_ref)
        sq_ref[...] = jnp.zeros_like(sq_ref)

      sum_ref[...] += jnp.sum(pre, axis=0, keepdims=True)
      sq_ref[...] += jnp.sum(pre * pre, axis=0, keepdims=True)

    @pl.when(ph == 1)
    def _normalize():
      inv_n = 1.0 / n
      mean = sum_ref[...] * inv_n
      var = sq_ref[...] * inv_n - mean * mean
      scale = g_ref[...] * lax.rsqrt(var + _EPS)
      shift = be_ref[...] - mean * scale
      pre = pre_ref[pl.ds(i * nb, nb), :]
      o_ref[...] = jnp.maximum(pre * scale + shift, 0.0)

  return pl.pallas_call(
      body,
      grid=(2, nblocks),
      in_specs=[
          pl.BlockSpec((2, nb, d_out), lambda p, i: (0, i, 0)),
          pl.BlockSpec((nb, d_out), lambda p, i: (i, 0)),
          pl.BlockSpec((nb, 1), lambda p, i: (i, 0)),
          pl.BlockSpec((1, d_out), lambda p, i: (0, 0)),
          pl.BlockSpec((1, d_out), lambda p, i: (0, 0)),
          pl.BlockSpec((1, d_out), lambda p, i: (0, 0)),
      ],
      out_specs=pl.BlockSpec((nb, d_out), lambda p, i: (i, 0)),
      out_shape=jax.ShapeDtypeStruct((n, d_out), jnp.float32),
      scratch_shapes=[
          pltpu.VMEM((n, d_out), jnp.float32),
          pltpu.VMEM((1, d_out), jnp.float32),
          pltpu.VMEM((1, d_out), jnp.float32),
      ],
  )(agg2, loop_msg, norm, bias, gamma, beta)


def kernel(x, edge_index, edge_type, norm, w_loop, w_bases, w_rel,
           w_bias, bn_gamma, bn_beta):
  n, d_in = x.shape
  e = edge_index.shape[1]
  n_bases, _, d_out = w_bases.shape
  n_rel = w_rel.shape[0]

  w_cat = jnp.concatenate([w_bases[b] for b in range(n_bases)], axis=1)

  workers = _NC * _NS
  gpw = -(-e // (_G * workers * 2)) * 2
  ng = gpw * workers
  e_pad = ng * _G
  pad = e_pad - e
  src = jnp.concatenate([edge_index[0], jnp.zeros((pad,), jnp.int32)])
  dst = jnp.concatenate([edge_index[1], jnp.zeros((pad,), jnp.int32)])
  et = jnp.concatenate([edge_type, jnp.full((pad,), n_rel, jnp.int32)])
  srcg = src.reshape(ng, _G)
  dstg = dst.reshape(ng, _G)
  etg = et.reshape(ng, _G)
  wr_rep = (jnp.zeros((_NREP, n_bases, _L), jnp.float32)
            .at[:n_rel].set(jnp.broadcast_to(w_rel[:, :, None],
                                             (n_rel, n_bases, _L)))
            .reshape(_NREP * n_bases * _L))

  n_pad = -(-n // (_G * _NS)) * (_G * _NS)
  h_cat, loop_msg = _matmul_call(
      x, w_cat, w_loop, n, d_in, n_bases * d_out, d_out, nb=2000)
  agg2 = _edge_call(h_cat, srcg, dstg, etg, wr_rep, n_pad, d_out, n_bases, gpw)
  out = _epilogue_call(
      agg2, loop_msg, norm, w_bias.reshape(1, d_out),
      bn_gamma.reshape(1, d_out), bn_beta.reshape(1, d_out), n, d_out,
      nb=2000)
  return out


# speedup vs baseline: 9.3242x; 1.8722x over previous
import functools

import jax
import jax.numpy as jnp
from jax import lax
from jax.experimental import pallas as pl
from jax.experimental.pallas import tpu as pltpu
from jax.experimental.pallas import tpu_sc as plsc

_NC = 2
_NS = 16
_L = 16
_G = 64
_NREP = 128
_EPS = 1e-5


def _matmul_call(x, w_cat, w_loop, n, d_in, d_cat, d_out, nb):
  def body(x_ref, wc_ref, wl_ref, h_ref, lm_ref):
    xv = x_ref[...]
    h = jnp.dot(xv, wc_ref[...], preferred_element_type=jnp.float32)
    bits = lax.bitcast_convert_type(h, jnp.int32)
    rnd = bits + 0x7FFF + ((bits >> 16) & 1)
    lo = lax.shift_right_logical(rnd[:, :d_cat // 2], 16)
    hi = rnd[:, d_cat // 2:] & jnp.int32(-65536)
    h_ref[...] = hi | lo
    lm_ref[...] = jnp.dot(xv, wl_ref[...], preferred_element_type=jnp.float32)

  grid = (n // nb,)
  return pl.pallas_call(
      body,
      grid=grid,
      in_specs=[
          pl.BlockSpec((nb, d_in), lambda i: (i, 0)),
          pl.BlockSpec((d_in, d_cat), lambda i: (0, 0)),
          pl.BlockSpec((d_in, d_out), lambda i: (0, 0)),
      ],
      out_specs=[
          pl.BlockSpec((nb, d_cat // 2), lambda i: (i, 0)),
          pl.BlockSpec((nb, d_out), lambda i: (i, 0)),
      ],
      out_shape=[
          jax.ShapeDtypeStruct((n, d_cat // 2), jnp.int32),
          jax.ShapeDtypeStruct((n, d_out), jnp.float32),
      ],
  )(x, w_cat, w_loop)


def _edge_call(h_cat, srcg, dstg, etg, wr_rep, n_pad, d_out, n_bases, gpw):
  d_cat = n_bases * d_out
  rows_ps = n_pad // _NS
  assert rows_ps % _G == 0
  nz = rows_ps // _G
  assert gpw % 2 == 0
  ng_total = gpw * _NC * _NS
  nvec = d_out // _L

  mesh = plsc.VectorSubcoreMesh(
      core_axis_name="c", subcore_axis_name="s",
      num_cores=_NC, num_subcores=_NS)

  @functools.partial(
      pl.kernel,
      out_type=jax.ShapeDtypeStruct((_NC, n_pad, d_out), jnp.float32),
      mesh=mesh,
      scratch_types=[
          [pltpu.VMEM((_G,), jnp.int32)] * 2,
          [pltpu.VMEM((_G,), jnp.int32)] * 2,
          [pltpu.VMEM((_G,), jnp.int32)] * 2,
          [pltpu.VMEM((_G, d_cat // 2), jnp.int32)] * 2,
          [pltpu.VMEM((_G, d_out), jnp.float32)] * 2,
          pltpu.VMEM((_NREP * n_bases * _L,), jnp.float32),
          pltpu.VMEM_SHARED((n_pad, d_out), jnp.float32),
          [pltpu.SemaphoreType.DMA] * 2,
          [pltpu.SemaphoreType.DMA] * 2,
          [pltpu.SemaphoreType.DMA] * 2,
          [pltpu.SemaphoreType.DMA] * 2,
      ],
  )
  def edge_kernel(h_hbm, src_hbm, dst_hbm, et_hbm, wr_hbm, out_hbm,
                  src_v, dst_v, et_v, rows_v, msg_v, wr_v, agg_sp,
                  isem, gsem, dsem, ssem):
    c = lax.axis_index("c")
    s = lax.axis_index("s")

    pltpu.sync_copy(wr_hbm, wr_v)

    def zrow(r, _):
      for j in range(d_out // _L):
        msg_v[0][r, pl.ds(j * _L, _L)] = jnp.zeros((_L,), jnp.float32)
      return 0
    lax.fori_loop(0, _G, zrow, 0)
    zbase = s * rows_ps
    for i in range(nz):
      pltpu.sync_copy(msg_v[0], agg_sp.at[pl.ds(zbase + i * _G, _G)])
    plsc.subcore_barrier()

    base_g = (c * _NS + s) * gpw
    last_g = ng_total - 1

    def issue_idx(b, gi):
      gi = jnp.minimum(gi, last_g)
      pltpu.async_copy(src_hbm.at[gi], src_v[b], isem[b])
      pltpu.async_copy(et_hbm.at[gi], et_v[b], isem[b])

    def wait_idx(b):
      pltpu.make_async_copy(src_hbm.at[0], src_v[b], isem[b]).wait()
      pltpu.make_async_copy(et_hbm.at[0], et_v[b], isem[b]).wait()

    def issue_gather(b):
      pltpu.async_copy(h_hbm.at[src_v[b]], rows_v[b], gsem[b])

    def wait_gather(b):
      pltpu.make_async_copy(h_hbm.at[src_v[b]], rows_v[b], gsem[b]).wait()

    def issue_dst(b, gi):
      gi = jnp.minimum(gi, last_g)
      pltpu.async_copy(dst_hbm.at[gi], dst_v[b], dsem[b])

    def wait_dst(b):
      pltpu.make_async_copy(dst_hbm.at[0], dst_v[b], dsem[b]).wait()

    def issue_scatter(b):
      pltpu.async_copy(msg_v[b], agg_sp.at[dst_v[b]], ssem[b], add=True)

    def wait_scatter(b):
      pltpu.make_async_copy(msg_v[b], agg_sp.at[dst_v[b]], ssem[b]).wait()

    def compute(b):
      rv = rows_v[b]
      mv = msg_v[b]

      def sub_body(k, _):
        e0 = k * _L
        et16 = et_v[b][pl.ds(e0, _L)]
        for l in range(_L):
          ei = e0 + l
          off = et16[l] * (n_bases * _L)
          cb = [wr_v[pl.ds(off + bb * _L, _L)] for bb in range(n_bases)]
          for m in range(nvec):
            vi = rv[ei, pl.ds(m * _L, _L)]
            p0 = lax.bitcast_convert_type(vi << 16, jnp.float32)
            p1 = lax.bitcast_convert_type(vi & jnp.int32(-65536), jnp.float32)
            mv[ei, pl.ds(m * _L, _L)] = cb[0] * p0 + cb[1] * p1
        return 0
      lax.fori_loop(0, _G // _L, sub_body, 0)

    def half_step(b, t, g):
      wait_idx(1 - b)
      issue_gather(1 - b)
      wait_gather(b)

      @pl.when(t > 0)
      def _drain():
        wait_scatter(b)
      issue_dst(b, g)
      compute(b)
      wait_dst(b)
      issue_scatter(b)
      issue_idx(b, g + 2)

    issue_idx(0, base_g)
    issue_idx(1, base_g + 1)
    wait_idx(0)
    issue_gather(0)

    def pipe_body(t, _):
      g = base_g + 2 * t
      half_step(0, t, g)
      half_step(1, t, g + 1)
      return 0
    lax.fori_loop(0, gpw // 2, pipe_body, 0)

    wait_scatter(0)
    wait_scatter(1)
    wait_gather(0)
    wait_idx(1)

    plsc.subcore_barrier()
    pltpu.sync_copy(agg_sp.at[pl.ds(zbase, rows_ps)],
                    out_hbm.at[c, pl.ds(zbase, rows_ps)])

  return edge_kernel(h_cat, srcg, dstg, etg, wr_rep)


def _epilogue_call(agg2, loop_msg, norm, bias, gamma, beta, n, d_out, nb):
  nblocks = n // nb

  def body(agg_ref, lm_ref, norm_ref, b_ref, g_ref, be_ref, o_ref,
           pre_ref, sum_ref, sq_ref):
    ph = pl.program_id(0)
    i = pl.program_id(1)

    @pl.when(ph == 0)
    def _compute():
      a = agg_ref[0] + agg_ref[1]
      pre = a * norm_ref[...] + b_ref[...] + lm_ref[...]
      pre_ref[pl.ds(i * nb, nb), :] = pre

      @pl.when(i == 0)
      def _init():
        sum_ref[...] = jnp.zeros_like(sum_ref)
        sq_ref[...] = jnp.zeros_like(sq_ref)

      sum_ref[...] += jnp.sum(pre, axis=0, keepdims=True)
      sq_ref[...] += jnp.sum(pre * pre, axis=0, keepdims=True)

    @pl.when(ph == 1)
    def _normalize():
      inv_n = 1.0 / n
      mean = sum_ref[...] * inv_n
      var = sq_ref[...] * inv_n - mean * mean
      scale = g_ref[...] * lax.rsqrt(var + _EPS)
      shift = be_ref[...] - mean * scale
      pre = pre_ref[pl.ds(i * nb, nb), :]
      o_ref[...] = jnp.maximum(pre * scale + shift, 0.0)

  return pl.pallas_call(
      body,
      grid=(2, nblocks),
      in_specs=[
          pl.BlockSpec((2, nb, d_out), lambda p, i: (0, i, 0)),
          pl.BlockSpec((nb, d_out), lambda p, i: (i, 0)),
          pl.BlockSpec((nb, 1), lambda p, i: (i, 0)),
          pl.BlockSpec((1, d_out), lambda p, i: (0, 0)),
          pl.BlockSpec((1, d_out), lambda p, i: (0, 0)),
          pl.BlockSpec((1, d_out), lambda p, i: (0, 0)),
      ],
      out_specs=pl.BlockSpec((nb, d_out), lambda p, i: (i, 0)),
      out_shape=jax.ShapeDtypeStruct((n, d_out), jnp.float32),
      scratch_shapes=[
          pltpu.VMEM((n, d_out), jnp.float32),
          pltpu.VMEM((1, d_out), jnp.float32),
          pltpu.VMEM((1, d_out), jnp.float32),
      ],
  )(agg2, loop_msg, norm, bias, gamma, beta)


def kernel(x, edge_index, edge_type, norm, w_loop, w_bases, w_rel,
           w_bias, bn_gamma, bn_beta):
  n, d_in = x.shape
  e = edge_index.shape[1]
  n_bases, _, d_out = w_bases.shape
  n_rel = w_rel.shape[0]

  assert n_bases == 2
  w_cat = jnp.concatenate([w_bases[b] for b in range(n_bases)], axis=1)

  workers = _NC * _NS
  gpw = -(-e // (_G * workers * 2)) * 2
  ng = gpw * workers
  e_pad = ng * _G
  pad = e_pad - e
  src = jnp.concatenate([edge_index[0], jnp.zeros((pad,), jnp.int32)])
  dst = jnp.concatenate([edge_index[1], jnp.zeros((pad,), jnp.int32)])
  et = jnp.concatenate([edge_type, jnp.full((pad,), n_rel, jnp.int32)])
  srcg = src.reshape(ng, _G)
  dstg = dst.reshape(ng, _G)
  etg = et.reshape(ng, _G)
  wr_rep = (jnp.zeros((_NREP, n_bases, _L), jnp.float32)
            .at[:n_rel].set(jnp.broadcast_to(w_rel[:, :, None],
                                             (n_rel, n_bases, _L)))
            .reshape(_NREP * n_bases * _L))

  n_pad = -(-n // (_G * _NS)) * (_G * _NS)
  h_cat, loop_msg = _matmul_call(
      x, w_cat, w_loop, n, d_in, n_bases * d_out, d_out, nb=2000)
  agg2 = _edge_call(h_cat, srcg, dstg, etg, wr_rep, n_pad, d_out, n_bases, gpw)
  out = _epilogue_call(
      agg2, loop_msg, norm, w_bias.reshape(1, d_out),
      bn_gamma.reshape(1, d_out), bn_beta.reshape(1, d_out), n, d_out,
      nb=2000)
  return out


# speedup vs baseline: 10.4980x; 1.1259x over previous
import functools

import jax
import jax.numpy as jnp
from jax import lax
from jax.experimental import pallas as pl
from jax.experimental.pallas import tpu as pltpu
from jax.experimental.pallas import tpu_sc as plsc

_NC = 2
_NS = 16
_L = 16
_G = 80
_NREP = 128
_EPS = 1e-5


def _matmul_call(x, w_cat, w_loop, n, d_in, d_cat, d_out, nb):
  def body(x_ref, wc_ref, wl_ref, h_ref, lm_ref):
    xv = x_ref[...]
    h = jnp.dot(xv, wc_ref[...], preferred_element_type=jnp.float32)
    bits = lax.bitcast_convert_type(h, jnp.int32)
    rnd = bits + 0x7FFF + ((bits >> 16) & 1)
    lo = lax.shift_right_logical(rnd[:, :d_cat // 2], 16)
    hi = rnd[:, d_cat // 2:] & jnp.int32(-65536)
    h_ref[...] = hi | lo
    lm_ref[...] = jnp.dot(xv, wl_ref[...], preferred_element_type=jnp.float32)

  grid = (n // nb,)
  return pl.pallas_call(
      body,
      grid=grid,
      in_specs=[
          pl.BlockSpec((nb, d_in), lambda i: (i, 0)),
          pl.BlockSpec((d_in, d_cat), lambda i: (0, 0)),
          pl.BlockSpec((d_in, d_out), lambda i: (0, 0)),
      ],
      out_specs=[
          pl.BlockSpec((nb, d_cat // 2), lambda i: (i, 0)),
          pl.BlockSpec((nb, d_out), lambda i: (i, 0)),
      ],
      out_shape=[
          jax.ShapeDtypeStruct((n, d_cat // 2), jnp.int32),
          jax.ShapeDtypeStruct((n, d_out), jnp.float32),
      ],
  )(x, w_cat, w_loop)


def _edge_call(h_cat, srcg, dstg, etg, wr_rep, n_pad, d_out, n_bases, gpw):
  d_cat = n_bases * d_out
  rows_ps = n_pad // _NS
  assert rows_ps % _G == 0
  nz = rows_ps // _G
  assert gpw % 2 == 0
  ng_total = gpw * _NC * _NS
  nvec = d_out // _L

  mesh = plsc.VectorSubcoreMesh(
      core_axis_name="c", subcore_axis_name="s",
      num_cores=_NC, num_subcores=_NS)

  @functools.partial(
      pl.kernel,
      out_type=jax.ShapeDtypeStruct((_NC, n_pad, d_out), jnp.float32),
      mesh=mesh,
      scratch_types=[
          [pltpu.VMEM((_G,), jnp.int32)] * 2,
          [pltpu.VMEM((_G,), jnp.int32)] * 2,
          [pltpu.VMEM((_G,), jnp.int32)] * 2,
          [pltpu.VMEM((_G, d_cat // 2), jnp.int32)] * 2,
          [pltpu.VMEM((_G, d_out), jnp.float32)] * 2,
          pltpu.VMEM((_NREP * n_bases * _L,), jnp.float32),
          pltpu.VMEM_SHARED((n_pad, d_out), jnp.float32),
          [pltpu.SemaphoreType.DMA] * 2,
          [pltpu.SemaphoreType.DMA] * 2,
          [pltpu.SemaphoreType.DMA] * 2,
          [pltpu.SemaphoreType.DMA] * 2,
      ],
  )
  def edge_kernel(h_hbm, src_hbm, dst_hbm, et_hbm, wr_hbm, out_hbm,
                  src_v, dst_v, et_v, rows_v, msg_v, wr_v, agg_sp,
                  isem, gsem, dsem, ssem):
    c = lax.axis_index("c")
    s = lax.axis_index("s")

    pltpu.sync_copy(wr_hbm, wr_v)

    def zrow(r, _):
      for j in range(d_out // _L):
        msg_v[0][r, pl.ds(j * _L, _L)] = jnp.zeros((_L,), jnp.float32)
      return 0
    lax.fori_loop(0, _G, zrow, 0)
    zbase = s * rows_ps
    for i in range(nz):
      pltpu.sync_copy(msg_v[0], agg_sp.at[pl.ds(zbase + i * _G, _G)])
    plsc.subcore_barrier()

    base_g = (c * _NS + s) * gpw
    last_g = ng_total - 1

    def issue_idx(b, gi):
      gi = jnp.minimum(gi, last_g)
      pltpu.async_copy(src_hbm.at[gi], src_v[b], isem[b])
      pltpu.async_copy(et_hbm.at[gi], et_v[b], isem[b])

    def wait_idx(b):
      pltpu.make_async_copy(src_hbm.at[0], src_v[b], isem[b]).wait()
      pltpu.make_async_copy(et_hbm.at[0], et_v[b], isem[b]).wait()

    def issue_gather(b):
      pltpu.async_copy(h_hbm.at[src_v[b]], rows_v[b], gsem[b])

    def wait_gather(b):
      pltpu.make_async_copy(h_hbm.at[src_v[b]], rows_v[b], gsem[b]).wait()

    def issue_dst(b, gi):
      gi = jnp.minimum(gi, last_g)
      pltpu.async_copy(dst_hbm.at[gi], dst_v[b], dsem[b])

    def wait_dst(b):
      pltpu.make_async_copy(dst_hbm.at[0], dst_v[b], dsem[b]).wait()

    def issue_scatter(b):
      pltpu.async_copy(msg_v[b], agg_sp.at[dst_v[b]], ssem[b], add=True)

    def wait_scatter(b):
      pltpu.make_async_copy(msg_v[b], agg_sp.at[dst_v[b]], ssem[b]).wait()

    def compute(b):
      rv = rows_v[b]
      mv = msg_v[b]

      def sub_body(k, _):
        e0 = k * _L
        et16 = et_v[b][pl.ds(e0, _L)]
        for l in range(_L):
          ei = e0 + l
          off = et16[l] * (n_bases * _L)
          cb = [wr_v[pl.ds(off + bb * _L, _L)] for bb in range(n_bases)]
          for m in range(nvec):
            vi = rv[ei, pl.ds(m * _L, _L)]
            p0 = lax.bitcast_convert_type(vi << 16, jnp.float32)
            p1 = lax.bitcast_convert_type(vi & jnp.int32(-65536), jnp.float32)
            mv[ei, pl.ds(m * _L, _L)] = cb[0] * p0 + cb[1] * p1
        return 0
      lax.fori_loop(0, _G // _L, sub_body, 0)

    def half_step(b, t, g):
      wait_idx(1 - b)
      issue_gather(1 - b)
      wait_gather(b)

      @pl.when(t > 0)
      def _drain():
        wait_scatter(b)
      issue_dst(b, g)
      compute(b)
      wait_dst(b)
      issue_scatter(b)
      issue_idx(b, g + 2)

    issue_idx(0, base_g)
    issue_idx(1, base_g + 1)
    wait_idx(0)
    issue_gather(0)

    def pipe_body(t, _):
      g = base_g + 2 * t
      half_step(0, t, g)
      half_step(1, t, g + 1)
      return 0
    lax.fori_loop(0, gpw // 2, pipe_body, 0)

    wait_scatter(0)
    wait_scatter(1)
    wait_gather(0)
    wait_idx(1)

    plsc.subcore_barrier()
    pltpu.sync_copy(agg_sp.at[pl.ds(zbase, rows_ps)],
                    out_hbm.at[c, pl.ds(zbase, rows_ps)])

  return edge_kernel(h_cat, srcg, dstg, etg, wr_rep)


def _epilogue_call(agg2, loop_msg, norm, bias, gamma, beta, n, d_out, nb):
  nblocks = n // nb

  def body(agg_ref, lm_ref, norm_ref, b_ref, g_ref, be_ref, o_ref,
           pre_ref, sum_ref, sq_ref):
    ph = pl.program_id(0)
    i = pl.program_id(1)

    @pl.when(ph == 0)
    def _compute():
      a = agg_ref[0] + agg_ref[1]
      pre = a * norm_ref[...] + b_ref[...] + lm_ref[...]
      pre_ref[pl.ds(i * nb, nb), :] = pre

      @pl.when(i == 0)
      def _init():
        sum_ref[...] = jnp.zeros_like(sum_ref)
        sq_ref[...] = jnp.zeros_like(sq_ref)

      sum_ref[...] += jnp.sum(pre, axis=0, keepdims=True)
      sq_ref[...] += jnp.sum(pre * pre, axis=0, keepdims=True)

    @pl.when(ph == 1)
    def _normalize():
      inv_n = 1.0 / n
      mean = sum_ref[...] * inv_n
      var = sq_ref[...] * inv_n - mean * mean
      scale = g_ref[...] * lax.rsqrt(var + _EPS)
      shift = be_ref[...] - mean * scale
      pre = pre_ref[pl.ds(i * nb, nb), :]
      o_ref[...] = jnp.maximum(pre * scale + shift, 0.0)

  return pl.pallas_call(
      body,
      grid=(2, nblocks),
      in_specs=[
          pl.BlockSpec((2, nb, d_out), lambda p, i: (0, i, 0)),
          pl.BlockSpec((nb, d_out), lambda p, i: (i, 0)),
          pl.BlockSpec((nb, 1), lambda p, i: (i, 0)),
          pl.BlockSpec((1, d_out), lambda p, i: (0, 0)),
          pl.BlockSpec((1, d_out), lambda p, i: (0, 0)),
          pl.BlockSpec((1, d_out), lambda p, i: (0, 0)),
      ],
      out_specs=pl.BlockSpec((nb, d_out), lambda p, i: (i, 0)),
      out_shape=jax.ShapeDtypeStruct((n, d_out), jnp.float32),
      scratch_shapes=[
          pltpu.VMEM((n, d_out), jnp.float32),
          pltpu.VMEM((1, d_out), jnp.float32),
          pltpu.VMEM((1, d_out), jnp.float32),
      ],
  )(agg2, loop_msg, norm, bias, gamma, beta)


def kernel(x, edge_index, edge_type, norm, w_loop, w_bases, w_rel,
           w_bias, bn_gamma, bn_beta):
  n, d_in = x.shape
  e = edge_index.shape[1]
  n_bases, _, d_out = w_bases.shape
  n_rel = w_rel.shape[0]

  assert n_bases == 2
  w_cat = jnp.concatenate([w_bases[b] for b in range(n_bases)], axis=1)

  workers = _NC * _NS
  gpw = -(-e // (_G * workers * 2)) * 2
  ng = gpw * workers
  e_pad = ng * _G
  pad = e_pad - e
  src = jnp.concatenate([edge_index[0], jnp.zeros((pad,), jnp.int32)])
  dst = jnp.concatenate([edge_index[1], jnp.zeros((pad,), jnp.int32)])
  et = jnp.concatenate([edge_type, jnp.full((pad,), n_rel, jnp.int32)])
  srcg = src.reshape(ng, _G)
  dstg = dst.reshape(ng, _G)
  etg = et.reshape(ng, _G)
  wr_rep = (jnp.zeros((_NREP, n_bases, _L), jnp.float32)
            .at[:n_rel].set(jnp.broadcast_to(w_rel[:, :, None],
                                             (n_rel, n_bases, _L)))
            .reshape(_NREP * n_bases * _L))

  n_pad = -(-n // (_G * _NS)) * (_G * _NS)
  h_cat, loop_msg = _matmul_call(
      x, w_cat, w_loop, n, d_in, n_bases * d_out, d_out, nb=2000)
  agg2 = _edge_call(h_cat, srcg, dstg, etg, wr_rep, n_pad, d_out, n_bases, gpw)
  out = _epilogue_call(
      agg2, loop_msg, norm, w_bias.reshape(1, d_out),
      bn_gamma.reshape(1, d_out), bn_beta.reshape(1, d_out), n, d_out,
      nb=2000)
  return out
